# baseline (speedup 1.0000x reference)
import functools

import jax
import jax.numpy as jnp
from jax import lax
from jax.experimental import pallas as pl
from jax.experimental.pallas import tpu as pltpu
from jax.experimental.pallas import tpu_sc as plsc

B, C, Hf, Wf = 4, 128, 256, 256
N = 4096
WWIN = 5
P = WWIN * WWIN
PPAD = 32
LPS = 16
PW = P * LPS
GRID = 64
NW = 32
NBT = N // NW
MCH = 8
NCH = NBT // MCH
CK = C // LPS


def _wtw_body(w_ref, m_ref):
    w = w_ref[...]
    m_ref[...] = lax.dot_general(w, w, (((0,), (0,)), ((), ())),
                                 preferred_element_type=jnp.float32)


def _wtw(w):
    return pl.pallas_call(
        _wtw_body,
        out_shape=jax.ShapeDtypeStruct((C, C), jnp.float32),
    )(w)


def _f0p_body(x_ref, m_ref, o_ref):
    x = x_ref[0]
    ix = lax.broadcasted_iota(jnp.int32, (Wf, GRID), 0)
    ik = lax.broadcasted_iota(jnp.int32, (Wf, GRID), 1)
    s = (ix == 4 * ik).astype(jnp.float32)
    t = lax.dot_general(s, x, (((0,), (1,)), ((), ())),
                        preferred_element_type=jnp.float32)
    o_ref[...] = jnp.dot(t, m_ref[...], preferred_element_type=jnp.float32)


def _f0p(f0, m):
    return pl.pallas_call(
        _f0p_body,
        grid=(B, GRID),
        in_specs=[
            pl.BlockSpec((1, C, Wf), lambda b, j: (b, 0, 4 * j)),
            pl.BlockSpec((C, C), lambda b, j: (0, 0)),
        ],
        out_specs=pl.BlockSpec((GRID, C), lambda b, j: (b * GRID + j, 0)),
        out_shape=jax.ShapeDtypeStruct((B * GRID * GRID, C), jnp.float32),
    )(f0, m)


_HB = 8


def _f1t_body(x_ref, o_ref):
    x = x_ref[0].reshape(C, _HB * Wf)
    o_ref[...] = x.T


def _f1t(f1):
    return pl.pallas_call(
        _f1t_body,
        grid=(B, Hf // _HB),
        in_specs=[pl.BlockSpec((1, C, _HB, Wf), lambda b, j: (b, 0, j, 0))],
        out_specs=pl.BlockSpec((_HB * Wf, C), lambda b, j: (b * (Hf // _HB) + j, 0)),
        out_shape=jax.ShapeDtypeStruct((B * Hf * Wf, C), jnp.float32),
    )(f1)


def _sc_corr_body(f0p, f1t, qidx, widx, pout,
                  qidx_v, widx_v, qm_buf, wbuf, pbuf, sem):
    wid = lax.axis_index("s") * 2 + lax.axis_index("c")
    base = wid * NBT
    pltpu.sync_copy(qidx.at[pl.ds(base, NBT)], qidx_v)
    pltpu.async_copy(f0p.at[qidx_v], qm_buf, sem).wait()
    pltpu.sync_copy(widx.at[pl.ds(base * P, NBT * P)], widx_v)

    def chunk(c, _):
        off = c * (MCH * P)
        cp0 = pltpu.async_copy(f1t.at[widx_v.at[pl.ds(off, 104)]],
                               wbuf.at[pl.ds(0, 104)], sem)
        cp1 = pltpu.async_copy(f1t.at[widx_v.at[pl.ds(off + 104, 96)]],
                               wbuf.at[pl.ds(104, 96)], sem)
        cp0.wait()
        cp1.wait()

        def dot_one(ni, _):
            n = c * MCH + ni
            q = [qm_buf[n, pl.ds(LPS * k, LPS)] for k in range(CK)]
            for p in range(P):
                row = ni * P + p
                v = q[0] * wbuf[row, pl.ds(0, LPS)]
                for k in range(1, CK):
                    v = v + q[k] * wbuf[row, pl.ds(LPS * k, LPS)]
                pbuf[ni, pl.ds(LPS * p, LPS)] = v
            return _

        lax.fori_loop(0, MCH, dot_one, None)
        pltpu.sync_copy(pbuf, pout.at[pl.ds(base + c * MCH, MCH)])
        return _

    lax.fori_loop(0, NCH, chunk, None)


@functools.lru_cache(maxsize=1)
def _sc_corr_kernel():
    return pl.kernel(
        _sc_corr_body,
        out_type=jax.ShapeDtypeStruct((N, PW), jnp.float32),
        mesh=plsc.VectorSubcoreMesh(core_axis_name="c", subcore_axis_name="s"),
        scratch_types=[pltpu.VMEM((NBT,), jnp.int32),
                       pltpu.VMEM((NBT * P,), jnp.int32),
                       pltpu.VMEM((NBT, C), jnp.float32),
                       pltpu.VMEM((MCH * P, C), jnp.float32),
                       pltpu.VMEM((MCH, PW), jnp.float32),
                       pltpu.SemaphoreType.DMA],
    )


def _finish_body(part_ref, m0_ref, m1_ref, mk0_ref, mk1_ref, off_ref):
    part = part_ref[...]
    jr = lax.broadcasted_iota(jnp.int32, (PW, PPAD), 0)
    pr = lax.broadcasted_iota(jnp.int32, (PW, PPAD), 1)
    sel = (jr // LPS == pr).astype(jnp.float32)
    corr = jnp.dot(part, sel, preferred_element_type=jnp.float32)
    m1 = m1_ref[...]
    pxy = m1 * 4
    px = pxy[:, 0:1]
    py = pxy[:, 1:2]
    lane = lax.broadcasted_iota(jnp.int32, (NBT, PPAD), 1)
    dx = lane % WWIN - WWIN // 2
    dy = lane // WWIN - WWIN // 2
    xs = px + dx
    ys = py + dy
    mask = (lane < P) & (xs >= 0) & (ys >= 0)
    corr = jnp.where(mask, corr, -1e9)
    mx = jnp.max(corr, axis=1, keepdims=True)
    e = jnp.exp(corr - mx)
    prob = e / jnp.sum(e, axis=1, keepdims=True)
    ex = jnp.sum(prob * xs.astype(jnp.float32), axis=1, keepdims=True)
    ey = jnp.sum(prob * ys.astype(jnp.float32), axis=1, keepdims=True)
    offs = jnp.concatenate([ex, ey], axis=1) - pxy.astype(jnp.float32)
    off_ref[...] = offs
    mk0_ref[...] = m0_ref[...].astype(jnp.float32) * 8.0
    mk1_ref[...] = m1.astype(jnp.float32) * 8.0 + offs * 2.0


def _finish(part, m0, m1):
    return pl.pallas_call(
        _finish_body,
        grid=(NW,),
        in_specs=[
            pl.BlockSpec((NBT, PW), lambda t: (t, 0)),
            pl.BlockSpec((NBT, 2), lambda t: (t, 0)),
            pl.BlockSpec((NBT, 2), lambda t: (t, 0)),
        ],
        out_specs=[
            pl.BlockSpec((NBT, 2), lambda t: (t, 0)),
            pl.BlockSpec((NBT, 2), lambda t: (t, 0)),
            pl.BlockSpec((NBT, 2), lambda t: (t, 0)),
        ],
        out_shape=[jax.ShapeDtypeStruct((N, 2), jnp.float32),
                   jax.ShapeDtypeStruct((N, 2), jnp.float32),
                   jax.ShapeDtypeStruct((N, 2), jnp.float32)],
    )(part, m0, m1)


def kernel(fine_feat0, fine_feat1, mkpts0_c, mkpts1_c, b_ids, W_proj):
    f1t = _f1t(fine_feat1)
    m = _wtw(W_proj)
    f0p = _f0p(fine_feat0.reshape(B, C, Hf * Wf), m)

    qidx = b_ids * (GRID * GRID) + mkpts0_c[:, 1] * GRID + mkpts0_c[:, 0]
    px = mkpts1_c[:, 0] * 4
    py = mkpts1_c[:, 1] * 4
    d = jnp.arange(-(WWIN // 2), WWIN // 2 + 1, dtype=jnp.int32)
    xc = jnp.maximum(px[:, None] + d[None, :], 0)
    yc = jnp.maximum(py[:, None] + d[None, :], 0)
    widx = (b_ids[:, None, None] * (Hf * Wf) + yc[:, :, None] * Wf
            + xc[:, None, :]).reshape(N * P)
    part = _sc_corr_kernel()(f0p, f1t, qidx.astype(jnp.int32),
                             widx.astype(jnp.int32))
    return _finish(part, mkpts0_c, mkpts1_c)


# speedup vs baseline: 1.8299x; 1.8299x over previous
import functools

import jax
import jax.numpy as jnp
from jax import lax
from jax.experimental import pallas as pl
from jax.experimental.pallas import tpu as pltpu
from jax.experimental.pallas import tpu_sc as plsc

B, C, Hf, Wf = 4, 128, 256, 256
N = 4096
WWIN = 5
P = WWIN * WWIN
PPAD = 32
GRID = 64
NW = 32
NBT = N // NW
MCH = 8
NCH = NBT // MCH
CK = C // 16


_HB = 8


def _prep_body(f0_ref, f1_ref, w_ref, f0p_ref, f1t_ref):
    x1 = f1_ref[0].reshape(C, _HB * Wf)
    f1t_ref[...] = x1.T
    w = w_ref[...]
    m = lax.dot_general(w, w, (((0,), (0,)), ((), ())),
                        preferred_element_type=jnp.float32)
    x0 = f0_ref[0]
    ix = lax.broadcasted_iota(jnp.int32, (Wf, GRID), 0)
    ik = lax.broadcasted_iota(jnp.int32, (Wf, GRID), 1)
    s = (ix == 4 * ik).astype(jnp.float32)
    t0 = lax.dot_general(s, x0[:, 0, :], (((0,), (1,)), ((), ())),
                         preferred_element_type=jnp.float32)
    t1 = lax.dot_general(s, x0[:, 4, :], (((0,), (1,)), ((), ())),
                         preferred_element_type=jnp.float32)
    t = jnp.concatenate([t0, t1], axis=0)
    f0p_ref[...] = jnp.dot(t, m, preferred_element_type=jnp.float32)


def _prep(f0, f1, w):
    return pl.pallas_call(
        _prep_body,
        grid=(B, Hf // _HB),
        in_specs=[
            pl.BlockSpec((1, C, _HB, Wf), lambda b, j: (b, 0, j, 0)),
            pl.BlockSpec((1, C, _HB, Wf), lambda b, j: (b, 0, j, 0)),
            pl.BlockSpec((C, C), lambda b, j: (0, 0)),
        ],
        out_specs=[
            pl.BlockSpec((2 * GRID, C), lambda b, j: (b * (Hf // _HB) + j, 0)),
            pl.BlockSpec((_HB * Wf, C), lambda b, j: (b * (Hf // _HB) + j, 0)),
        ],
        out_shape=[jax.ShapeDtypeStruct((B * GRID * GRID, C), jnp.float32),
                   jax.ShapeDtypeStruct((B * Hf * Wf, C), jnp.float32)],
    )(f0, f1, w)


_GDN = lax.GatherDimensionNumbers(offset_dims=(), collapsed_slice_dims=(0,),
                                  start_index_map=(0,))


def _rot16(v, r):
    idx = (lax.iota(jnp.int32, 16) + r) & 15
    return lax.gather(v, idx[:, None], _GDN, (1,),
                      mode=lax.GatherScatterMode.PROMISE_IN_BOUNDS)


def _sc_corr_body(f0p, f1t, qidx, widx, corr_out,
                  qidx_v, widx_v, qm_buf, wbuf, corr_buf, sem):
    wid = lax.axis_index("s") * 2 + lax.axis_index("c")
    base = wid * NBT
    pltpu.sync_copy(qidx.at[pl.ds(base, NBT)], qidx_v)
    pltpu.async_copy(f0p.at[qidx_v], qm_buf, sem).wait()
    pltpu.sync_copy(widx.at[pl.ds(base * P, NBT * P)], widx_v)
    li16 = lax.iota(jnp.int32, 16)

    def chunk(c, _):
        off = c * (MCH * P)
        pltpu.async_copy(f1t.at[widx_v.at[pl.ds(off, 104)]],
                         wbuf.at[pl.ds(0, 104)], sem).wait()
        pltpu.async_copy(f1t.at[widx_v.at[pl.ds(off + 104, 96)]],
                         wbuf.at[pl.ds(104, 96)], sem).wait()

        def dot_one(ni, _):
            n = c * MCH + ni
            q = [qm_buf[n, pl.ds(16 * k, 16)] for k in range(CK)]
            acc0 = jnp.zeros((16,), jnp.float32)
            acc1 = jnp.zeros((16,), jnp.float32)
            for p in range(P):
                row = ni * P + p
                v = q[0] * wbuf[row, pl.ds(0, 16)]
                for k in range(1, CK):
                    v = v + q[k] * wbuf[row, pl.ds(16 * k, 16)]
                for sh in (8, 4, 2, 1):
                    v = v + _rot16(v, sh)
                if p < 16:
                    acc0 = jnp.where(li16 == p, v, acc0)
                else:
                    acc1 = jnp.where(li16 == (p - 16), v, acc1)
            corr_buf[n, pl.ds(0, 16)] = acc0
            corr_buf[n, pl.ds(16, 16)] = acc1
            return _

        lax.fori_loop(0, MCH, dot_one, None)
        return _

    lax.fori_loop(0, NCH, chunk, None)
    pltpu.sync_copy(corr_buf, corr_out.at[pl.ds(base, NBT)])


@functools.lru_cache(maxsize=1)
def _sc_corr_kernel():
    return pl.kernel(
        _sc_corr_body,
        out_type=jax.ShapeDtypeStruct((N, PPAD), jnp.float32),
        mesh=plsc.VectorSubcoreMesh(core_axis_name="c", subcore_axis_name="s"),
        scratch_types=[pltpu.VMEM((NBT,), jnp.int32),
                       pltpu.VMEM((NBT * P,), jnp.int32),
                       pltpu.VMEM((NBT, C), jnp.float32),
                       pltpu.VMEM((MCH * P, C), jnp.float32),
                       pltpu.VMEM((NBT, PPAD), jnp.float32),
                       pltpu.SemaphoreType.DMA],
    )


def _finish_body(corr_ref, m0_ref, m1_ref, mk0_ref, mk1_ref, off_ref):
    corr = corr_ref[...]
    m1 = m1_ref[...]
    pxy = m1 * 4
    px = pxy[:, 0:1]
    py = pxy[:, 1:2]
    lane = lax.broadcasted_iota(jnp.int32, (NBT, PPAD), 1)
    dx = lane % WWIN - WWIN // 2
    dy = lane // WWIN - WWIN // 2
    xs = px + dx
    ys = py + dy
    mask = (lane < P) & (xs >= 0) & (ys >= 0)
    corr = jnp.where(mask, corr, -1e9)
    mx = jnp.max(corr, axis=1, keepdims=True)
    e = jnp.exp(corr - mx)
    prob = e / jnp.sum(e, axis=1, keepdims=True)
    ex = jnp.sum(prob * xs.astype(jnp.float32), axis=1, keepdims=True)
    ey = jnp.sum(prob * ys.astype(jnp.float32), axis=1, keepdims=True)
    offs = jnp.concatenate([ex, ey], axis=1) - pxy.astype(jnp.float32)
    off_ref[...] = offs
    mk0_ref[...] = m0_ref[...].astype(jnp.float32) * 8.0
    mk1_ref[...] = m1.astype(jnp.float32) * 8.0 + offs * 2.0


def _finish(corr, m0, m1):
    return pl.pallas_call(
        _finish_body,
        grid=(NW,),
        in_specs=[
            pl.BlockSpec((NBT, PPAD), lambda t: (t, 0)),
            pl.BlockSpec((NBT, 2), lambda t: (t, 0)),
            pl.BlockSpec((NBT, 2), lambda t: (t, 0)),
        ],
        out_specs=[
            pl.BlockSpec((NBT, 2), lambda t: (t, 0)),
            pl.BlockSpec((NBT, 2), lambda t: (t, 0)),
            pl.BlockSpec((NBT, 2), lambda t: (t, 0)),
        ],
        out_shape=[jax.ShapeDtypeStruct((N, 2), jnp.float32),
                   jax.ShapeDtypeStruct((N, 2), jnp.float32),
                   jax.ShapeDtypeStruct((N, 2), jnp.float32)],
    )(corr, m0, m1)


def kernel(fine_feat0, fine_feat1, mkpts0_c, mkpts1_c, b_ids, W_proj):
    f0p, f1t = _prep(fine_feat0, fine_feat1, W_proj)

    qidx = b_ids * (GRID * GRID) + mkpts0_c[:, 1] * GRID + mkpts0_c[:, 0]
    px = mkpts1_c[:, 0] * 4
    py = mkpts1_c[:, 1] * 4
    d = jnp.arange(-(WWIN // 2), WWIN // 2 + 1, dtype=jnp.int32)
    xc = jnp.maximum(px[:, None] + d[None, :], 0)
    yc = jnp.maximum(py[:, None] + d[None, :], 0)
    widx = (b_ids[:, None, None] * (Hf * Wf) + yc[:, :, None] * Wf
            + xc[:, None, :]).reshape(N * P)

    corr = _sc_corr_kernel()(f0p, f1t, qidx.astype(jnp.int32),
                             widx.astype(jnp.int32))
    return _finish(corr, mkpts0_c, mkpts1_c)


# speedup vs baseline: 1.8966x; 1.0365x over previous
import functools

import jax
import jax.numpy as jnp
from jax import lax
from jax.experimental import pallas as pl
from jax.experimental.pallas import tpu as pltpu
from jax.experimental.pallas import tpu_sc as plsc

B, C, Hf, Wf = 4, 128, 256, 256
N = 4096
WWIN = 5
P = WWIN * WWIN
PPAD = 32
GRID = 64
NW = 32
NBT = N // NW
MCH = 8
NCH = NBT // MCH
CK = C // 16


_HB = 8


def _prep_body(f0_ref, f1_ref, w_ref, f0p_ref, f1t_ref):
    x1 = f1_ref[0].reshape(C, _HB * Wf)
    f1t_ref[...] = x1.T
    w = w_ref[...]
    m = lax.dot_general(w, w, (((0,), (0,)), ((), ())),
                        preferred_element_type=jnp.float32)
    x0 = f0_ref[0]
    ix = lax.broadcasted_iota(jnp.int32, (Wf, GRID), 0)
    ik = lax.broadcasted_iota(jnp.int32, (Wf, GRID), 1)
    s = (ix == 4 * ik).astype(jnp.float32)
    t0 = lax.dot_general(s, x0[:, 0, :], (((0,), (1,)), ((), ())),
                         preferred_element_type=jnp.float32)
    t1 = lax.dot_general(s, x0[:, 4, :], (((0,), (1,)), ((), ())),
                         preferred_element_type=jnp.float32)
    t = jnp.concatenate([t0, t1], axis=0)
    f0p_ref[...] = jnp.dot(t, m, preferred_element_type=jnp.float32)


def _prep(f0, f1, w):
    return pl.pallas_call(
        _prep_body,
        grid=(B, Hf // _HB),
        in_specs=[
            pl.BlockSpec((1, C, _HB, Wf), lambda b, j: (b, 0, j, 0)),
            pl.BlockSpec((1, C, _HB, Wf), lambda b, j: (b, 0, j, 0)),
            pl.BlockSpec((C, C), lambda b, j: (0, 0)),
        ],
        out_specs=[
            pl.BlockSpec((2 * GRID, C), lambda b, j: (b * (Hf // _HB) + j, 0)),
            pl.BlockSpec((_HB * Wf, C), lambda b, j: (b * (Hf // _HB) + j, 0)),
        ],
        out_shape=[jax.ShapeDtypeStruct((B * GRID * GRID, C), jnp.float32),
                   jax.ShapeDtypeStruct((B * Hf * Wf, C), jnp.float32)],
    )(f0, f1, w)


_GDN = lax.GatherDimensionNumbers(offset_dims=(), collapsed_slice_dims=(0,),
                                  start_index_map=(0,))


def _rot16(v, r):
    idx = (lax.iota(jnp.int32, 16) + r) & 15
    return lax.gather(v, idx[:, None], _GDN, (1,),
                      mode=lax.GatherScatterMode.PROMISE_IN_BOUNDS)


def _sc_corr_body(f0p, f1t, qidx, widx, corr_out,
                  qidx_v, widx_v, qm_buf, wbuf, corr_buf, sem):
    wid = lax.axis_index("s") * 2 + lax.axis_index("c")
    base = wid * NBT
    pltpu.sync_copy(qidx.at[pl.ds(base, NBT)], qidx_v)
    pltpu.async_copy(f0p.at[qidx_v], qm_buf, sem).wait()
    pltpu.sync_copy(widx.at[pl.ds(base * P, NBT * P)], widx_v)
    li16 = lax.iota(jnp.int32, 16)

    def chunk(c, _):
        off = c * (MCH * P)
        cp0 = pltpu.async_copy(f1t.at[widx_v.at[pl.ds(off, 104)]],
                               wbuf.at[pl.ds(0, 104)], sem)
        cp1 = pltpu.async_copy(f1t.at[widx_v.at[pl.ds(off + 104, 96)]],
                               wbuf.at[pl.ds(104, 96)], sem)
        cp0.wait()
        cp1.wait()

        def dot_one(ni, _):
            n = c * MCH + ni
            q = [qm_buf[n, pl.ds(16 * k, 16)] for k in range(CK)]
            acc0 = jnp.zeros((16,), jnp.float32)
            acc1 = jnp.zeros((16,), jnp.float32)
            for p in range(P):
                row = ni * P + p
                v = q[0] * wbuf[row, pl.ds(0, 16)]
                for k in range(1, CK):
                    v = v + q[k] * wbuf[row, pl.ds(16 * k, 16)]
                for sh in (8, 4, 2, 1):
                    v = v + _rot16(v, sh)
                if p < 16:
                    acc0 = jnp.where(li16 == p, v, acc0)
                else:
                    acc1 = jnp.where(li16 == (p - 16), v, acc1)
            corr_buf[n, pl.ds(0, 16)] = acc0
            corr_buf[n, pl.ds(16, 16)] = acc1
            return _

        lax.fori_loop(0, MCH, dot_one, None)
        return _

    lax.fori_loop(0, NCH, chunk, None)
    pltpu.sync_copy(corr_buf, corr_out.at[pl.ds(base, NBT)])


@functools.lru_cache(maxsize=1)
def _sc_corr_kernel():
    return pl.kernel(
        _sc_corr_body,
        out_type=jax.ShapeDtypeStruct((N, PPAD), jnp.float32),
        mesh=plsc.VectorSubcoreMesh(core_axis_name="c", subcore_axis_name="s"),
        scratch_types=[pltpu.VMEM((NBT,), jnp.int32),
                       pltpu.VMEM((NBT * P,), jnp.int32),
                       pltpu.VMEM((NBT, C), jnp.float32),
                       pltpu.VMEM((MCH * P, C), jnp.float32),
                       pltpu.VMEM((NBT, PPAD), jnp.float32),
                       pltpu.SemaphoreType.DMA],
    )


def _finish_body(corr_ref, m0_ref, m1_ref, mk0_ref, mk1_ref, off_ref):
    corr = corr_ref[...]
    m1 = m1_ref[...]
    pxy = m1 * 4
    px = pxy[:, 0:1]
    py = pxy[:, 1:2]
    lane = lax.broadcasted_iota(jnp.int32, (NBT, PPAD), 1)
    dx = lane % WWIN - WWIN // 2
    dy = lane // WWIN - WWIN // 2
    xs = px + dx
    ys = py + dy
    mask = (lane < P) & (xs >= 0) & (ys >= 0)
    corr = jnp.where(mask, corr, -1e9)
    mx = jnp.max(corr, axis=1, keepdims=True)
    e = jnp.exp(corr - mx)
    prob = e / jnp.sum(e, axis=1, keepdims=True)
    ex = jnp.sum(prob * xs.astype(jnp.float32), axis=1, keepdims=True)
    ey = jnp.sum(prob * ys.astype(jnp.float32), axis=1, keepdims=True)
    offs = jnp.concatenate([ex, ey], axis=1) - pxy.astype(jnp.float32)
    off_ref[...] = offs
    mk0_ref[...] = m0_ref[...].astype(jnp.float32) * 8.0
    mk1_ref[...] = m1.astype(jnp.float32) * 8.0 + offs * 2.0


def _finish(corr, m0, m1):
    return pl.pallas_call(
        _finish_body,
        grid=(NW,),
        in_specs=[
            pl.BlockSpec((NBT, PPAD), lambda t: (t, 0)),
            pl.BlockSpec((NBT, 2), lambda t: (t, 0)),
            pl.BlockSpec((NBT, 2), lambda t: (t, 0)),
        ],
        out_specs=[
            pl.BlockSpec((NBT, 2), lambda t: (t, 0)),
            pl.BlockSpec((NBT, 2), lambda t: (t, 0)),
            pl.BlockSpec((NBT, 2), lambda t: (t, 0)),
        ],
        out_shape=[jax.ShapeDtypeStruct((N, 2), jnp.float32),
                   jax.ShapeDtypeStruct((N, 2), jnp.float32),
                   jax.ShapeDtypeStruct((N, 2), jnp.float32)],
    )(corr, m0, m1)


def kernel(fine_feat0, fine_feat1, mkpts0_c, mkpts1_c, b_ids, W_proj):
    f0p, f1t = _prep(fine_feat0, fine_feat1, W_proj)

    qidx = b_ids * (GRID * GRID) + mkpts0_c[:, 1] * GRID + mkpts0_c[:, 0]
    px = mkpts1_c[:, 0] * 4
    py = mkpts1_c[:, 1] * 4
    d = jnp.arange(-(WWIN // 2), WWIN // 2 + 1, dtype=jnp.int32)
    xc = jnp.maximum(px[:, None] + d[None, :], 0)
    yc = jnp.maximum(py[:, None] + d[None, :], 0)
    widx = (b_ids[:, None, None] * (Hf * Wf) + yc[:, :, None] * Wf
            + xc[:, None, :]).reshape(N * P)

    corr = _sc_corr_kernel()(f0p, f1t, qidx.astype(jnp.int32),
                             widx.astype(jnp.int32))
    return _finish(corr, mkpts0_c, mkpts1_c)
